# baseline (speedup 1.0000x reference)
import functools
import math

import jax
import jax.numpy as jnp
from jax import lax
from jax.experimental import pallas as pl
from jax.experimental.pallas import tpu as pltpu
from jax.experimental.pallas import tpu_sc as plsc

D_MODEL = 64
NUM_WORKERS = 32
CHUNK = 128


SPLIT = 524288
BLKW = 8192


def _tc_transform(table_t, w_t8, b8):
    V = table_t.shape[1]
    nblk = SPLIT // BLKW
    last_blk = (V + BLKW - 1) // BLKW - 1

    def body(lo_ref, hi_ref, w_ref, b_ref, o_ref):
        dn = (((0,), (0,)), ((), ()))
        lo = lax.dot_general(lo_ref[...], w_ref[...], dn,
                             preferred_element_type=jnp.float32) + b_ref[...]
        hi = lax.dot_general(hi_ref[...], w_ref[...], dn,
                             preferred_element_type=jnp.float32) + b_ref[...]
        o_ref[...] = jnp.concatenate([lo, hi], axis=1)

    return pl.pallas_call(
        body,
        grid=(nblk,),
        in_specs=[
            pl.BlockSpec((D_MODEL, BLKW), lambda i: (0, i)),
            pl.BlockSpec(
                (D_MODEL, BLKW),
                lambda i, n=nblk, lb=last_blk: (0, jnp.minimum(i + n, lb)),
            ),
            pl.BlockSpec((D_MODEL, D_MODEL), lambda i: (0, 0)),
            pl.BlockSpec((1, D_MODEL), lambda i: (0, 0)),
        ],
        out_specs=pl.BlockSpec((BLKW, 2 * D_MODEL), lambda i: (i, 0)),
        out_shape=jax.ShapeDtypeStruct((SPLIT, 2 * D_MODEL), jnp.float32),
    )(table_t, table_t, w_t8, b8)


def _sc_gather(table, idx2, bsz, seqlen):
    chunk = idx2.shape[1]
    nb = bsz // NUM_WORKERS

    mesh = plsc.VectorSubcoreMesh(core_axis_name="c", subcore_axis_name="s")

    @functools.partial(
        pl.kernel,
        out_type=jax.ShapeDtypeStruct((bsz * seqlen, D_MODEL), jnp.float32),
        mesh=mesh,
        scratch_types=[
            pltpu.VMEM((2 * nb, chunk), jnp.int32),
            pltpu.VMEM((seqlen, D_MODEL), jnp.float32),
            pltpu.SemaphoreType.DMA,
        ],
        compiler_params=pltpu.CompilerParams(use_tc_tiling_on_sc=False),
    )
    def gather_kernel(table_hbm, idx_hbm, out_hbm, idx_v, stage_v, sem):
        wid = lax.axis_index("s") * 2 + lax.axis_index("c")
        pltpu.sync_copy(idx_hbm.at[pl.ds(wid * 2 * nb, 2 * nb)], idx_v)

        def body(lb, carry):
            cp0 = pltpu.async_copy(
                table_hbm.at[idx_v.at[2 * lb]],
                stage_v.at[pl.ds(0, chunk)], sem)
            cp1 = pltpu.async_copy(
                table_hbm.at[idx_v.at[2 * lb + 1]],
                stage_v.at[pl.ds(chunk, chunk)], sem)
            cp0.wait()
            cp1.wait()
            pltpu.sync_copy(
                stage_v, out_hbm.at[pl.ds((wid * nb + lb) * seqlen, seqlen)])
            return carry

        lax.fori_loop(0, nb, body, 0)

    return gather_kernel(table, idx2)


def _tc_to_output_layout(y3, bsz, seqlen):
    BL = 128
    AW = 10
    cols = AW * 2 * D_MODEL
    y6 = y3.reshape(bsz, seqlen * D_MODEL)

    def body(x_ref, o_ref):
        eye = jnp.eye(BL, dtype=jnp.float32)
        xt = lax.dot_general(
            x_ref[...], eye, (((0,), (0,)), ((), ())),
            preferred_element_type=jnp.float32)
        o_ref[...] = xt.reshape(2 * AW, D_MODEL, BL)

    zt = pl.pallas_call(
        body,
        grid=(bsz // BL, (seqlen // 2) // AW),
        in_specs=[pl.BlockSpec((BL, cols), lambda j, k: (j, k))],
        out_specs=pl.BlockSpec((2 * AW, D_MODEL, BL), lambda j, k: (k, 0, j)),
        out_shape=jax.ShapeDtypeStruct((seqlen, D_MODEL, bsz), jnp.float32),
    )(y6)
    return zt.transpose(2, 0, 1)


def kernel(x, table, W, b, lang_id):
    bsz, seqlen = x.shape

    scale = math.sqrt(float(D_MODEL))
    w_t8 = W.T * scale
    b8 = (b * scale).reshape(1, D_MODEL)

    t2_pairs = _tc_transform(table.T, w_t8, b8)
    t2 = t2_pairs.reshape(2 * SPLIT, D_MODEL)

    xi = x.reshape(-1).astype(jnp.int32)
    g = jnp.where(xi < SPLIT, 2 * xi, 2 * (xi - SPLIT) + 1)
    idx2 = g.reshape(bsz * 2, seqlen // 2)

    y3 = _sc_gather(t2, idx2, bsz, seqlen)
    return _tc_to_output_layout(y3, bsz, seqlen)


# speedup vs baseline: 1.0111x; 1.0111x over previous
import functools
import math

import jax
import jax.numpy as jnp
from jax import lax
from jax.experimental import pallas as pl
from jax.experimental.pallas import tpu as pltpu
from jax.experimental.pallas import tpu_sc as plsc

D_MODEL = 64
NUM_WORKERS = 32
CHUNK = 128


SPLIT = 524288
BLKW = 8192


def _tc_transform(table_t, w_t8, b8):
    V = table_t.shape[1]
    nblk = SPLIT // BLKW
    last_blk = (V + BLKW - 1) // BLKW - 1

    def body(lo_ref, hi_ref, w_ref, b_ref, o_ref):
        dn = (((0,), (0,)), ((), ()))
        lo = lax.dot_general(lo_ref[...], w_ref[...], dn,
                             preferred_element_type=jnp.float32) + b_ref[...]
        hi = lax.dot_general(hi_ref[...], w_ref[...], dn,
                             preferred_element_type=jnp.float32) + b_ref[...]
        o_ref[...] = jnp.concatenate([lo, hi], axis=1)

    return pl.pallas_call(
        body,
        grid=(nblk,),
        in_specs=[
            pl.BlockSpec((D_MODEL, BLKW), lambda i: (0, i)),
            pl.BlockSpec(
                (D_MODEL, BLKW),
                lambda i, n=nblk, lb=last_blk: (0, jnp.minimum(i + n, lb)),
            ),
            pl.BlockSpec((D_MODEL, D_MODEL), lambda i: (0, 0)),
            pl.BlockSpec((1, D_MODEL), lambda i: (0, 0)),
        ],
        out_specs=pl.BlockSpec((BLKW, 2 * D_MODEL), lambda i: (i, 0)),
        out_shape=jax.ShapeDtypeStruct((SPLIT, 2 * D_MODEL), jnp.float32),
    )(table_t, table_t, w_t8, b8)


def _sc_gather(table, idx2, bsz, seqlen):
    chunk = idx2.shape[1]
    nb = bsz // NUM_WORKERS

    mesh = plsc.VectorSubcoreMesh(core_axis_name="c", subcore_axis_name="s")

    @functools.partial(
        pl.kernel,
        out_type=jax.ShapeDtypeStruct((bsz * seqlen, D_MODEL), jnp.float32),
        mesh=mesh,
        scratch_types=[
            pltpu.VMEM((2 * nb, chunk), jnp.int32),
            pltpu.VMEM((seqlen, D_MODEL), jnp.float32),
            pltpu.SemaphoreType.DMA,
        ],
        compiler_params=pltpu.CompilerParams(use_tc_tiling_on_sc=False),
    )
    def gather_kernel(table_hbm, idx_hbm, out_hbm, idx_v, stage_v, sem):
        wid = lax.axis_index("s") * 2 + lax.axis_index("c")
        pltpu.sync_copy(idx_hbm.at[pl.ds(wid * 2 * nb, 2 * nb)], idx_v)

        def body(lb, carry):
            cp0 = pltpu.async_copy(
                table_hbm.at[idx_v.at[2 * lb]],
                stage_v.at[pl.ds(0, chunk)], sem)
            cp1 = pltpu.async_copy(
                table_hbm.at[idx_v.at[2 * lb + 1]],
                stage_v.at[pl.ds(chunk, chunk)], sem)
            cp0.wait()
            cp1.wait()
            pltpu.sync_copy(
                stage_v, out_hbm.at[pl.ds((wid * nb + lb) * seqlen, seqlen)])
            return carry

        lax.fori_loop(0, nb, body, 0)

    return gather_kernel(table, idx2)


def _tc_to_output_layout(y3, bsz, seqlen):
    BL = 128
    AW = 10
    cols = AW * 2 * D_MODEL
    y6 = y3.reshape(bsz, seqlen * D_MODEL)

    def body(x_ref, o_ref):
        xt = x_ref[...].T
        o_ref[...] = xt.reshape(2 * AW, D_MODEL, BL)

    zt = pl.pallas_call(
        body,
        grid=(bsz // BL, (seqlen // 2) // AW),
        in_specs=[pl.BlockSpec((BL, cols), lambda j, k: (j, k))],
        out_specs=pl.BlockSpec((2 * AW, D_MODEL, BL), lambda j, k: (k, 0, j)),
        out_shape=jax.ShapeDtypeStruct((seqlen, D_MODEL, bsz), jnp.float32),
    )(y6)
    return zt.transpose(2, 0, 1)


def kernel(x, table, W, b, lang_id):
    bsz, seqlen = x.shape

    scale = math.sqrt(float(D_MODEL))
    w_t8 = W.T * scale
    b8 = (b * scale).reshape(1, D_MODEL)

    t2_pairs = _tc_transform(table.T, w_t8, b8)
    t2 = t2_pairs.reshape(2 * SPLIT, D_MODEL)

    xi = x.reshape(-1).astype(jnp.int32)
    g = jnp.where(xi < SPLIT, 2 * xi, 2 * (xi - SPLIT) + 1)
    idx2 = g.reshape(bsz * 2, seqlen // 2)

    y3 = _sc_gather(t2, idx2, bsz, seqlen)
    return _tc_to_output_layout(y3, bsz, seqlen)
